# baseline (speedup 1.0000x reference)
import functools

import numpy as np
import jax
import jax.numpy as jnp
from jax import lax
from jax.experimental import pallas as pl
from jax.experimental.pallas import tpu as pltpu
from jax.experimental.pallas import tpu_sc as plsc

_VOCAB = 1000000
_EMBED = 64
_SEQ = 50
_BATCH = 16384

_NC = 2
_NS = 16
_NW = _NC * _NS

_VFULL = _VOCAB // 128 * 128
_NGRP = _VFULL // 128
_CH_PER_W = 124

_NBT = _BATCH // 128
_BT_PER_W = _NBT // _NW
_LS = 2
_NCH = _BT_PER_W * (_SEQ // _LS)


def _positional_encoding(seq_len, d_model):
    pos = np.arange(seq_len)[:, np.newaxis]
    i = np.arange(d_model)[np.newaxis, :]
    angle_rates = 1.0 / np.power(10000, 2 * (i // 2) / np.float32(d_model))
    angle_rads = pos * angle_rates
    angle_rads[:, 0::2] = np.sin(angle_rads[:, 0::2])
    angle_rads[:, 1::2] = np.cos(angle_rads[:, 1::2])
    return angle_rads.astype(np.float32)


def _fmt_body(tt_hbm, tail_hbm, lin_hbm,
              tin0, tin1, tout0, isem0, isem1, osem0, osem1):
    c = lax.axis_index("c")
    s = lax.axis_index("s")
    wid = s * _NC + c

    iota = lax.iota(jnp.int32, 16)
    iotah = iota // 2
    ov = (iota % 2) * 64

    def gbase(ci):
        return lax.rem((wid * _CH_PER_W + ci) * 2, _NGRP)

    def fire_in(ci, tin, isem):
        return pltpu.async_copy(
            tt_hbm.at[:, pl.ds(gbase(ci) * 128, 256)], tin, isem)

    def transpose(tin, tout):
        @plsc.parallel_loop(0, 64, unroll=2)
        def _(e):
            cols = ov + e
            for m in range(16):
                vals = tin[e, pl.ds(m * 16, 16)]
                plsc.store_scatter(tout, [iotah + 8 * m, cols], vals)

    fire_in(0, tin0, isem0)
    fire_in(1, tin1, isem1)

    def loop_body(j, carry):
        for par, (tin, isem) in enumerate(((tin0, isem0), (tin1, isem1))):
            ci = 2 * j + par
            pltpu.make_async_copy(
                tt_hbm.at[:, pl.ds(0, 256)], tin, isem).wait()
            @pl.when(j + par >= 1)
            def _():
                pltpu.make_async_copy(
                    tout0.at[:, pl.ds(0, 128)],
                    lin_hbm.at[pl.ds(0, 128)], osem0).wait()
            transpose(tin, tout0)
            pltpu.async_copy(
                tout0.at[:, pl.ds(0, 128)],
                lin_hbm.at[pl.ds(gbase(ci) * 64, 128)], osem0)
            fire_in(ci + 2, tin, isem)
        return carry

    lax.fori_loop(0, _CH_PER_W // 2, loop_body, 0)

    pltpu.make_async_copy(tt_hbm.at[:, pl.ds(0, 256)], tin0, isem0).wait()
    pltpu.make_async_copy(tt_hbm.at[:, pl.ds(0, 256)], tin1, isem1).wait()
    pltpu.make_async_copy(
        tout0.at[:, pl.ds(0, 128)], lin_hbm.at[pl.ds(0, 128)], osem0).wait()

    @pl.when(wid == 0)
    def _():
        pltpu.async_copy(tail_hbm, tout0.at[pl.ds(0, 32), pl.ds(0, 128)],
                         isem0).wait()
        pltpu.sync_copy(tout0.at[pl.ds(0, 32), pl.ds(0, 128)],
                        lin_hbm.at[pl.ds(_VFULL // 2, 32)])


def _emb_body(xt_hbm, pos_hbm, table_hbm, out_hbm,
              idx0, idx1, pos_v,
              ra0, rb0, ra1, rb1, oa0, ob0, oa1, ob1,
              gsem0, gsem1, osem0, osem1, isem0, isem1):
    c = lax.axis_index("c")
    s = lax.axis_index("s")
    wid = s * _NC + c

    iota = lax.iota(jnp.int32, 16)
    zero16 = jnp.zeros((16,), jnp.int32)
    rowg = [iota + 16 * g for g in range(8)]

    def chunk_lb(ch):
        chm = lax.rem(ch, _NCH)
        q = chm // (_SEQ // _LS)
        l0 = lax.rem(chm, _SEQ // _LS) * _LS
        bt = wid * _BT_PER_W + q
        return l0, bt

    def fire_idx(ch, idx_v, isem):
        l0, bt = chunk_lb(ch)
        return pltpu.async_copy(
            xt_hbm.at[pl.ds(l0, _LS), pl.ds(bt * 128, 128)], idx_v, isem)

    def fire_gathers(idx_v, ra, rb, gsem):
        pltpu.async_copy(table_hbm.at[idx_v.at[0]], ra, gsem)
        pltpu.async_copy(table_hbm.at[idx_v.at[1]], rb, gsem)

    etv = [(16 * k + iota) // 8 for k in range(4)]
    erv = [(16 * k + iota) % 8 for k in range(4)]

    def add_transpose(l, rows_v, outb_v):
        pk = [pos_v[l, pl.ds(16 * k, 16)] for k in range(4)]

        @plsc.parallel_loop(0, 128, unroll=2)
        def _(br):
            colv = zero16 + br
            for k in range(4):
                vals = rows_v[br, pl.ds(k * 16, 16)] + pk[k]
                plsc.store_scatter(
                    outb_v, [zero16, etv[k], zero16, erv[k], colv], vals)

    def out_slice(l, bt):
        return out_hbm.at[pl.ds(l, 1), pl.ds(0, 8), pl.ds(bt, 1)]

    pltpu.sync_copy(pos_hbm, pos_v)

    fire_idx(0, idx0, isem0).wait()
    fire_idx(1, idx1, isem1).wait()
    fire_gathers(idx0, ra0, rb0, gsem0)
    fire_gathers(idx1, ra1, rb1, gsem1)

    def loop_body(j, carry):
        for par, (idx_v, ra, rb, oa, ob, gsem, osem, isem) in enumerate((
                (idx0, ra0, rb0, oa0, ob0, gsem0, osem0, isem0),
                (idx1, ra1, rb1, oa1, ob1, gsem1, osem1, isem1))):
            ch = 2 * j + par
            l0, bt = chunk_lb(ch)
            pltpu.make_async_copy(table_hbm.at[pl.ds(0, 128)], ra, gsem).wait()
            pltpu.make_async_copy(table_hbm.at[pl.ds(0, 128)], rb, gsem).wait()
            ih = fire_idx(ch + 2, idx_v, isem)
            @pl.when(j >= 1)
            def _():
                pltpu.make_async_copy(oa.at[:, :, :, :, pl.ds(0, 128)],
                                      out_slice(0, 0), osem).wait()
                pltpu.make_async_copy(ob.at[:, :, :, :, pl.ds(0, 128)],
                                      out_slice(0, 0), osem).wait()
            add_transpose(l0, ra, oa)
            add_transpose(l0 + 1, rb, ob)
            pltpu.async_copy(oa.at[:, :, :, :, pl.ds(0, 128)],
                             out_slice(l0, bt), osem)
            pltpu.async_copy(ob.at[:, :, :, :, pl.ds(0, 128)],
                             out_slice(l0 + 1, bt), osem)
            ih.wait()
            fire_gathers(idx_v, ra, rb, gsem)
        return carry

    lax.fori_loop(0, _NCH // 2, loop_body, 0)

    for ra, rb, oa, ob, gsem, osem in (
            (ra0, rb0, oa0, ob0, gsem0, osem0),
            (ra1, rb1, oa1, ob1, gsem1, osem1)):
        pltpu.make_async_copy(oa.at[:, :, :, :, pl.ds(0, 128)],
                              out_slice(0, 0), osem).wait()
        pltpu.make_async_copy(ob.at[:, :, :, :, pl.ds(0, 128)],
                              out_slice(0, 0), osem).wait()
        pltpu.make_async_copy(table_hbm.at[pl.ds(0, 128)], ra, gsem).wait()
        pltpu.make_async_copy(table_hbm.at[pl.ds(0, 128)], rb, gsem).wait()


@functools.partial(jax.jit, static_argnames=())
def kernel(x, table):
    pos = jnp.asarray(_positional_encoding(_SEQ, _EMBED))
    tableT = table.T
    tail = table[_VFULL:].reshape(32, 128)
    xT = x.T
    mesh = plsc.VectorSubcoreMesh(core_axis_name="c", subcore_axis_name="s")

    fmt = pl.kernel(
        _fmt_body,
        out_type=jax.ShapeDtypeStruct((_VOCAB // 2, 128), jnp.float32),
        mesh=mesh,
        scratch_types=[
            pltpu.VMEM((64, 256), jnp.float32),
            pltpu.VMEM((64, 256), jnp.float32),
            pltpu.VMEM((128, 129), jnp.float32),
            pltpu.SemaphoreType.DMA,
            pltpu.SemaphoreType.DMA,
            pltpu.SemaphoreType.DMA,
            pltpu.SemaphoreType.DMA,
        ],
        compiler_params=pltpu.CompilerParams(
            use_tc_tiling_on_sc=True, needs_layout_passes=False),
    )
    lin2 = fmt(tableT, tail)
    table_lin = lin2.reshape(_VOCAB, _EMBED)

    emb = pl.kernel(
        _emb_body,
        out_type=jax.ShapeDtypeStruct((_SEQ, 8, 128, 8, 128), jnp.float32),
        mesh=mesh,
        scratch_types=[
            pltpu.VMEM((_LS, 128), jnp.int32),
            pltpu.VMEM((_LS, 128), jnp.int32),
            pltpu.VMEM((_SEQ, _EMBED), jnp.float32),
            pltpu.VMEM((128, _EMBED), jnp.float32),
            pltpu.VMEM((128, _EMBED), jnp.float32),
            pltpu.VMEM((128, _EMBED), jnp.float32),
            pltpu.VMEM((128, _EMBED), jnp.float32),
            pltpu.VMEM((1, 8, 1, 8, 129), jnp.float32),
            pltpu.VMEM((1, 8, 1, 8, 129), jnp.float32),
            pltpu.VMEM((1, 8, 1, 8, 129), jnp.float32),
            pltpu.VMEM((1, 8, 1, 8, 129), jnp.float32),
            pltpu.SemaphoreType.DMA,
            pltpu.SemaphoreType.DMA,
            pltpu.SemaphoreType.DMA,
            pltpu.SemaphoreType.DMA,
            pltpu.SemaphoreType.DMA,
            pltpu.SemaphoreType.DMA,
        ],
        compiler_params=pltpu.CompilerParams(
            use_tc_tiling_on_sc=False, needs_layout_passes=False),
    )
    out5 = emb(xT, pos, table_lin)
    return out5.transpose(2, 4, 0, 1, 3).reshape(_BATCH, _SEQ, _EMBED)


# speedup vs baseline: 1.4573x; 1.4573x over previous
import functools

import numpy as np
import jax
import jax.numpy as jnp
from jax import lax
from jax.experimental import pallas as pl
from jax.experimental.pallas import tpu as pltpu
from jax.experimental.pallas import tpu_sc as plsc

_VOCAB = 1000000
_EMBED = 64
_SEQ = 50
_BATCH = 16384

_NC = 2
_NS = 16
_NW = _NC * _NS

_VFULL = _VOCAB // 128 * 128
_NGRP = _VFULL // 128
_CH_PER_W = 124

_NBT = _BATCH // 128
_BT_PER_W = _NBT // _NW
_LS = 2
_NCH = _BT_PER_W * (_SEQ // _LS)


def _positional_encoding(seq_len, d_model):
    pos = np.arange(seq_len)[:, np.newaxis]
    i = np.arange(d_model)[np.newaxis, :]
    angle_rates = 1.0 / np.power(10000, 2 * (i // 2) / np.float32(d_model))
    angle_rads = pos * angle_rates
    angle_rads[:, 0::2] = np.sin(angle_rads[:, 0::2])
    angle_rads[:, 1::2] = np.cos(angle_rads[:, 1::2])
    return angle_rads.astype(np.float32)


def _fmt_body(tt_hbm, tail_hbm, lin_hbm,
              tin0, tin1, tout0, isem0, isem1, osem0, osem1):
    c = lax.axis_index("c")
    s = lax.axis_index("s")
    wid = s * _NC + c

    iota = lax.iota(jnp.int32, 16)
    iotah = iota // 2
    ov = (iota % 2) * 64

    def gbase(ci):
        return lax.rem((wid * _CH_PER_W + ci) * 2, _NGRP)

    def fire_in(ci, tin, isem):
        return pltpu.async_copy(
            tt_hbm.at[:, pl.ds(gbase(ci) * 128, 256)], tin, isem)

    def transpose(tin, tout):
        @plsc.parallel_loop(0, 64, unroll=2)
        def _(e):
            cols = ov + e
            for m in range(16):
                vals = tin[e, pl.ds(m * 16, 16)]
                plsc.store_scatter(tout, [iotah + 8 * m, cols], vals)

    fire_in(0, tin0, isem0)
    fire_in(1, tin1, isem1)

    def loop_body(j, carry):
        for par, (tin, isem) in enumerate(((tin0, isem0), (tin1, isem1))):
            ci = 2 * j + par
            pltpu.make_async_copy(
                tt_hbm.at[:, pl.ds(0, 256)], tin, isem).wait()
            @pl.when(j + par >= 1)
            def _():
                pltpu.make_async_copy(
                    tout0.at[:, pl.ds(0, 128)],
                    lin_hbm.at[pl.ds(0, 128)], osem0).wait()
            transpose(tin, tout0)
            pltpu.async_copy(
                tout0.at[:, pl.ds(0, 128)],
                lin_hbm.at[pl.ds(gbase(ci) * 64, 128)], osem0)
            fire_in(ci + 2, tin, isem)
        return carry

    lax.fori_loop(0, _CH_PER_W // 2, loop_body, 0)

    pltpu.make_async_copy(tt_hbm.at[:, pl.ds(0, 256)], tin0, isem0).wait()
    pltpu.make_async_copy(tt_hbm.at[:, pl.ds(0, 256)], tin1, isem1).wait()
    pltpu.make_async_copy(
        tout0.at[:, pl.ds(0, 128)], lin_hbm.at[pl.ds(0, 128)], osem0).wait()

    @pl.when(wid == 0)
    def _():
        pltpu.async_copy(tail_hbm, tout0.at[pl.ds(0, 32), pl.ds(0, 128)],
                         isem0).wait()
        pltpu.sync_copy(tout0.at[pl.ds(0, 32), pl.ds(0, 128)],
                        lin_hbm.at[pl.ds(_VFULL // 2, 32)])


def _emb_body(xt_hbm, pos_hbm, table_hbm, out_hbm,
              idx0, idx1, pos_v,
              ra0, rb0, ra1, rb1, oa0, ob0, oa1, ob1,
              gsem0, gsem1, osem0, osem1, isem0, isem1):
    c = lax.axis_index("c")
    s = lax.axis_index("s")
    wid = s * _NC + c

    iota = lax.iota(jnp.int32, 16)
    zero16 = jnp.zeros((16,), jnp.int32)
    rowg = [iota + 16 * g for g in range(8)]

    def chunk_lb(ch):
        chm = lax.rem(ch, _NCH)
        q = chm // (_SEQ // _LS)
        l0 = lax.rem(chm, _SEQ // _LS) * _LS
        bt = wid * _BT_PER_W + q
        return l0, bt

    def fire_idx(ch, idx_v, isem):
        l0, bt = chunk_lb(ch)
        return pltpu.async_copy(
            xt_hbm.at[pl.ds(l0, _LS), pl.ds(bt * 128, 128)], idx_v, isem)

    def fire_gathers(idx_v, ra, rb, gsem):
        pltpu.async_copy(table_hbm.at[idx_v.at[0]], ra, gsem)
        pltpu.async_copy(table_hbm.at[idx_v.at[1]], rb, gsem)

    etv = [(16 * k + iota) // 8 for k in range(4)]
    erv = [(16 * k + iota) % 8 for k in range(4)]

    def add_transpose(l, rows_v, outb_v):
        pk = [pos_v[l, pl.ds(16 * k, 16)] for k in range(4)]

        @plsc.parallel_loop(0, 128, unroll=2)
        def _(br):
            colv = zero16 + br
            for k in range(4):
                vals = rows_v[br, pl.ds(k * 16, 16)] + pk[k]
                plsc.store_scatter(
                    outb_v, [zero16, etv[k], zero16, erv[k], colv], vals)

    def out_slice(l, bt):
        return out_hbm.at[pl.ds(l, 1), pl.ds(0, 8), pl.ds(bt, 1)]

    pltpu.sync_copy(pos_hbm, pos_v)

    fire_idx(0, idx0, isem0).wait()
    fire_idx(1, idx1, isem1).wait()
    fire_gathers(idx0, ra0, rb0, gsem0)
    fire_gathers(idx1, ra1, rb1, gsem1)

    def loop_body(j, carry):
        for par, (idx_v, ra, rb, oa, ob, gsem, osem, isem) in enumerate((
                (idx0, ra0, rb0, oa0, ob0, gsem0, osem0, isem0),
                (idx1, ra1, rb1, oa1, ob1, gsem1, osem1, isem1))):
            ch = 2 * j + par
            l0, bt = chunk_lb(ch)
            pltpu.make_async_copy(table_hbm.at[pl.ds(0, 128)], ra, gsem).wait()
            pltpu.make_async_copy(table_hbm.at[pl.ds(0, 128)], rb, gsem).wait()
            ih = fire_idx(ch + 2, idx_v, isem)
            @pl.when(j >= 1)
            def _():
                pltpu.make_async_copy(oa.at[:, :, :, :, pl.ds(0, 128)],
                                      out_slice(0, 0), osem).wait()
                pltpu.make_async_copy(ob.at[:, :, :, :, pl.ds(0, 128)],
                                      out_slice(0, 0), osem).wait()
            add_transpose(l0, ra, oa)
            add_transpose(l0 + 1, rb, ob)
            pltpu.async_copy(oa.at[:, :, :, :, pl.ds(0, 128)],
                             out_slice(l0, bt), osem)
            pltpu.async_copy(ob.at[:, :, :, :, pl.ds(0, 128)],
                             out_slice(l0 + 1, bt), osem)
            ih.wait()
            fire_gathers(idx_v, ra, rb, gsem)
        return carry

    lax.fori_loop(0, _NCH // 2, loop_body, 0)

    for ra, rb, oa, ob, gsem, osem in (
            (ra0, rb0, oa0, ob0, gsem0, osem0),
            (ra1, rb1, oa1, ob1, gsem1, osem1)):
        pltpu.make_async_copy(oa.at[:, :, :, :, pl.ds(0, 128)],
                              out_slice(0, 0), osem).wait()
        pltpu.make_async_copy(ob.at[:, :, :, :, pl.ds(0, 128)],
                              out_slice(0, 0), osem).wait()
        pltpu.make_async_copy(table_hbm.at[pl.ds(0, 128)], ra, gsem).wait()
        pltpu.make_async_copy(table_hbm.at[pl.ds(0, 128)], rb, gsem).wait()


@functools.partial(jax.jit, static_argnames=())
def kernel(x, table):
    pos = jnp.asarray(_positional_encoding(_SEQ, _EMBED))
    tableT = table.T
    tail = table[_VFULL:].reshape(32, 128)
    xT = x.T
    mesh = plsc.VectorSubcoreMesh(core_axis_name="c", subcore_axis_name="s")

    fmt = pl.kernel(
        _fmt_body,
        out_type=jax.ShapeDtypeStruct((_VOCAB // 2, 128), jnp.float32),
        mesh=mesh,
        scratch_types=[
            pltpu.VMEM((64, 256), jnp.float32),
            pltpu.VMEM((64, 256), jnp.float32),
            pltpu.VMEM((128, 129), jnp.float32),
            pltpu.SemaphoreType.DMA,
            pltpu.SemaphoreType.DMA,
            pltpu.SemaphoreType.DMA,
            pltpu.SemaphoreType.DMA,
        ],
        compiler_params=pltpu.CompilerParams(
            use_tc_tiling_on_sc=True, needs_layout_passes=False),
    )
    table_lin = table

    emb = pl.kernel(
        _emb_body,
        out_type=jax.ShapeDtypeStruct((_SEQ, 8, 128, 8, 128), jnp.float32),
        mesh=mesh,
        scratch_types=[
            pltpu.VMEM((_LS, 128), jnp.int32),
            pltpu.VMEM((_LS, 128), jnp.int32),
            pltpu.VMEM((_SEQ, _EMBED), jnp.float32),
            pltpu.VMEM((128, _EMBED), jnp.float32),
            pltpu.VMEM((128, _EMBED), jnp.float32),
            pltpu.VMEM((128, _EMBED), jnp.float32),
            pltpu.VMEM((128, _EMBED), jnp.float32),
            pltpu.VMEM((1, 8, 1, 8, 129), jnp.float32),
            pltpu.VMEM((1, 8, 1, 8, 129), jnp.float32),
            pltpu.VMEM((1, 8, 1, 8, 129), jnp.float32),
            pltpu.VMEM((1, 8, 1, 8, 129), jnp.float32),
            pltpu.SemaphoreType.DMA,
            pltpu.SemaphoreType.DMA,
            pltpu.SemaphoreType.DMA,
            pltpu.SemaphoreType.DMA,
            pltpu.SemaphoreType.DMA,
            pltpu.SemaphoreType.DMA,
        ],
        compiler_params=pltpu.CompilerParams(
            use_tc_tiling_on_sc=False, needs_layout_passes=False),
    )
    out5 = emb(xT, pos, table_lin)
    return out5.transpose(2, 4, 0, 1, 3).reshape(_BATCH, _SEQ, _EMBED)
